# baseline (speedup 1.0000x reference)
import functools

import jax
import jax.numpy as jnp
from jax.experimental import pallas as pl
from jax.experimental.pallas import tpu as pltpu

TOKENS_PER_BLOCK = 2048
NUM_EXPERTS = 64


def _router_block(x_ref, w_ref, b_ref, w_out_ref, i_out_ref):
    xb = x_ref[...]
    wb = w_ref[...]
    logits = jax.lax.dot_general(
        xb, wb, (((1,), (1,)), ((), ())),
        preferred_element_type=jnp.float32)
    logits = logits + b_ref[...]

    t = logits.shape[0]
    eidx = jax.lax.broadcasted_iota(jnp.int32, (t, NUM_EXPERTS), 1)
    m1 = jnp.max(logits, axis=1, keepdims=True)
    i1 = jnp.argmax(logits, axis=1).reshape(t, 1).astype(jnp.int32)
    masked = jnp.where(eidx == i1, -jnp.inf, logits)
    m2 = jnp.max(masked, axis=1, keepdims=True)
    i2 = jnp.argmax(masked, axis=1).reshape(t, 1).astype(jnp.int32)

    s = jnp.exp(m2 - m1)
    w1 = 1.0 / (1.0 + s)
    w2 = s / (1.0 + s)

    w_out_ref[...] = jnp.concatenate([w1, w2], axis=1)
    i_out_ref[...] = jnp.concatenate([i1, i2], axis=1)


@functools.partial(jax.jit, static_argnames=())
def kernel(x, W, b):
    d = x.shape[-1]
    xt = x.reshape(-1, d)
    n = xt.shape[0]
    t = TOKENS_PER_BLOCK
    grid = (n // t,)

    weights, indices = pl.pallas_call(
        _router_block,
        grid=grid,
        in_specs=[
            pl.BlockSpec((t, d), lambda i: (i, 0)),
            pl.BlockSpec((NUM_EXPERTS, d), lambda i: (0, 0)),
            pl.BlockSpec((1, NUM_EXPERTS), lambda i: (0, 0)),
        ],
        out_specs=[
            pl.BlockSpec((t, 2), lambda i: (i, 0)),
            pl.BlockSpec((t, 2), lambda i: (i, 0)),
        ],
        out_shape=[
            jax.ShapeDtypeStruct((n, 2), jnp.float32),
            jax.ShapeDtypeStruct((n, 2), jnp.int32),
        ],
        compiler_params=pltpu.CompilerParams(
            dimension_semantics=("parallel",),
        ),
    )(xt, W, b.reshape(1, NUM_EXPERTS))
    return (weights, indices)
